# Initial kernel scaffold; baseline (speedup 1.0000x reference)
import jax
import jax.numpy as jnp
from jax.experimental import pallas as pl


def kernel(x, table):
    raise NotImplementedError("write your pallas kernel here")


# speedup vs baseline: 1.8764x; 1.8764x over previous
import functools

import jax
import jax.numpy as jnp
from jax import lax
from jax.experimental import pallas as pl
from jax.experimental.pallas import tpu as pltpu
from jax.experimental.pallas import tpu_sc as plsc

VOCAB = 1000000
DIM = 64
ROWS = 16384
COLS = 50
B = ROWS * COLS

_INFO = plsc.get_sparse_core_info()
NC = _INFO.num_cores
NS = _INFO.num_subcores
NW = NC * NS
B_PER_W = B // NW

CH = 128
N_CHUNKS = B_PER_W // CH
NBUF = 4


def _body(x_hbm, table_hbm, out_hbm, idx_v, rows_v, gsems):
    wid = lax.axis_index("s") * NC + lax.axis_index("c")
    base = wid * B_PER_W

    pltpu.sync_copy(x_hbm.at[wid], idx_v)

    def fire(j, b):
        pltpu.async_copy(table_hbm.at[idx_v.at[j]], rows_v.at[b], gsems.at[b])

    def drain_store(j, b):
        pltpu.make_async_copy(table_hbm.at[idx_v.at[j]], rows_v.at[b],
                              gsems.at[b]).wait()
        pltpu.sync_copy(rows_v.at[b], out_hbm.at[pl.ds(base + j * CH, CH)])

    for b in range(NBUF):
        fire(b, b)

    @pl.loop(0, N_CHUNKS - NBUF, step=NBUF)
    def _(g):
        for b in range(NBUF):
            j = g + b
            drain_store(j, b)
            fire(j + NBUF, b)

    for b in range(NBUF):
        drain_store(N_CHUNKS - NBUF + b, b)


@jax.jit
def _sc_gather(x3, table):
    k = pl.kernel(
        _body,
        out_type=jax.ShapeDtypeStruct((B, DIM), jnp.float32),
        mesh=plsc.VectorSubcoreMesh(core_axis_name="c", subcore_axis_name="s"),
        scratch_types=[
            pltpu.VMEM((N_CHUNKS, CH), jnp.int32),
            pltpu.VMEM((NBUF, CH, DIM), jnp.float32),
            pltpu.SemaphoreType.DMA((NBUF,)),
        ],
        compiler_params=pltpu.CompilerParams(use_tc_tiling_on_sc=False),
    )
    return k(x3, table)


def kernel(x, table):
    x3 = x.reshape(NW, N_CHUNKS, CH).astype(jnp.int32)
    out = _sc_gather(x3, table)
    return out.reshape(ROWS, COLS, DIM)


# speedup vs baseline: 1.8781x; 1.0009x over previous
import jax
import jax.numpy as jnp
from jax import lax
from jax.experimental import pallas as pl
from jax.experimental.pallas import tpu as pltpu
from jax.experimental.pallas import tpu_sc as plsc

VOCAB = 1000000
DIM = 64
ROWS = 16384
COLS = 50
B = ROWS * COLS

_INFO = plsc.get_sparse_core_info()
NC = _INFO.num_cores
NS = _INFO.num_subcores
NW = NC * NS
B_PER_W = B // NW

CH = 128
N_CHUNKS = B_PER_W // CH
NBUF = 8
GDEPTH = 6


def _body(x_hbm, table_hbm, out_hbm, idx_v, rows_v, gsems, ssems):
    wid = lax.axis_index("s") * NC + lax.axis_index("c")
    base = wid * B_PER_W

    pltpu.sync_copy(x_hbm.at[wid], idx_v)

    def g_fire(j, b):
        pltpu.async_copy(table_hbm.at[idx_v.at[j]], rows_v.at[b], gsems.at[b])

    def g_wait(j, b):
        pltpu.make_async_copy(table_hbm.at[idx_v.at[j]], rows_v.at[b],
                              gsems.at[b]).wait()

    def s_fire(j, b):
        pltpu.async_copy(rows_v.at[b], out_hbm.at[pl.ds(base + j * CH, CH)],
                         ssems.at[b])

    def s_wait(j, b):
        pltpu.make_async_copy(rows_v.at[b],
                              out_hbm.at[pl.ds(base + j * CH, CH)],
                              ssems.at[b]).wait()

    for j in range(GDEPTH):
        g_fire(j, j % NBUF)

    for j in range(NBUF):
        g_wait(j, j % NBUF)
        s_fire(j, j % NBUF)
        if j >= NBUF - GDEPTH:
            s_wait(j - (NBUF - GDEPTH), (j - (NBUF - GDEPTH)) % NBUF)
        g_fire(j + GDEPTH, (j + GDEPTH) % NBUF)

    @pl.loop(NBUF, N_CHUNKS - NBUF, step=NBUF)
    def _(g):
        for b in range(NBUF):
            j = g + b
            g_wait(j, b)
            s_fire(j, b)
            s_wait(j - (NBUF - GDEPTH), (b - (NBUF - GDEPTH)) % NBUF)
            g_fire(j + GDEPTH, (b + GDEPTH) % NBUF)

    for j in range(N_CHUNKS - NBUF, N_CHUNKS):
        g_wait(j, j % NBUF)
        s_fire(j, j % NBUF)
        s_wait(j - (NBUF - GDEPTH), (j - (NBUF - GDEPTH)) % NBUF)
        if j + GDEPTH < N_CHUNKS:
            g_fire(j + GDEPTH, (j + GDEPTH) % NBUF)

    for j in range(N_CHUNKS - (NBUF - GDEPTH), N_CHUNKS):
        s_wait(j, j % NBUF)


@jax.jit
def _sc_gather(x3, table):
    k = pl.kernel(
        _body,
        out_type=jax.ShapeDtypeStruct((B, DIM), jnp.float32),
        mesh=plsc.VectorSubcoreMesh(core_axis_name="c", subcore_axis_name="s"),
        scratch_types=[
            pltpu.VMEM((N_CHUNKS, CH), jnp.int32),
            pltpu.VMEM((NBUF, CH, DIM), jnp.float32),
            pltpu.SemaphoreType.DMA((NBUF,)),
            pltpu.SemaphoreType.DMA((NBUF,)),
        ],
        compiler_params=pltpu.CompilerParams(use_tc_tiling_on_sc=False),
    )
    return k(x3, table)


def kernel(x, table):
    x3 = x.reshape(NW, N_CHUNKS, CH).astype(jnp.int32)
    out = _sc_gather(x3, table)
    return out.reshape(ROWS, COLS, DIM)
